# Initial kernel scaffold; baseline (speedup 1.0000x reference)
import jax
import jax.numpy as jnp
from jax.experimental import pallas as pl


def kernel(inputs, edge_index, W_self0, W_neigh0, b0, W_self1, W_neigh1, b1, W_self2, W_neigh2, b2):
    raise NotImplementedError("write your pallas kernel here")


# speedup vs baseline: 3.1142x; 3.1142x over previous
import functools

import jax
import jax.numpy as jnp
from jax import lax
from jax.experimental import pallas as pl
from jax.experimental.pallas import tpu as pltpu, tpu_sc as plsc

N = 10000
E = 320000
D = 128
NPAD = 10240
NW = 32
CHUNK = 128
CHUNKS = 80
EPAD = NW * CHUNKS * CHUNK
ROWS_PER_TILE = NPAD // 16
DEGW = 16

_MESH = plsc.VectorSubcoreMesh(core_axis_name="c", subcore_axis_name="s")

GRP = 8
NGRP = CHUNKS // GRP


def _sc_agg_body(p_hbm, src_hbm, dst_hbm, z128, agg_out,
                 src_g, dst_g, rows0, rows1, agg_sh, sem0, sem1):
    c = lax.axis_index("c")
    s = lax.axis_index("s")
    wid = s * 2 + c
    r0 = s * ROWS_PER_TILE

    pltpu.sync_copy(z128.at[pl.ds(r0, ROWS_PER_TILE)],
                    agg_sh.at[pl.ds(r0, ROWS_PER_TILE)])
    plsc.subcore_barrier()

    def group(g, carry):
        pltpu.sync_copy(src_hbm.at[wid, pl.ds(g * GRP, GRP)], src_g)
        pltpu.sync_copy(dst_hbm.at[wid, pl.ds(g * GRP, GRP)], dst_g)
        pltpu.async_copy(p_hbm.at[src_g.at[0]], rows0, sem0)

        def pair(j, carry2):
            a = 2 * j
            b = a + 1
            pltpu.async_copy(p_hbm.at[src_g.at[b]], rows1, sem1)
            pltpu.make_async_copy(p_hbm.at[src_g.at[a]], rows0, sem0).wait()
            pltpu.sync_copy(rows0, agg_sh.at[dst_g.at[a]], add=True)

            @pl.when(j < GRP // 2 - 1)
            def _():
                pltpu.async_copy(p_hbm.at[src_g.at[a + 2]], rows0, sem0)

            pltpu.make_async_copy(p_hbm.at[src_g.at[b]], rows1, sem1).wait()
            pltpu.sync_copy(rows1, agg_sh.at[dst_g.at[b]], add=True)
            return carry2

        lax.fori_loop(0, GRP // 2, pair, 0)
        return carry

    lax.fori_loop(0, NGRP, group, 0)

    plsc.subcore_barrier()

    pltpu.sync_copy(agg_sh.at[pl.ds(r0, ROWS_PER_TILE)],
                    agg_out.at[c, pl.ds(r0, ROWS_PER_TILE)])


_sc_agg = pl.kernel(
    _sc_agg_body,
    out_type=(jax.ShapeDtypeStruct((2, NPAD, D), jnp.float32),),
    mesh=_MESH,
    scratch_types=[
        pltpu.VMEM((GRP, CHUNK), jnp.int32),
        pltpu.VMEM((GRP, CHUNK), jnp.int32),
        pltpu.VMEM((CHUNK, D), jnp.float32),
        pltpu.VMEM((CHUNK, D), jnp.float32),
        pltpu.VMEM_SHARED((NPAD, D), jnp.float32),
        pltpu.SemaphoreType.DMA,
        pltpu.SemaphoreType.DMA,
    ],
    name="sage_sc_agg",
)


def _sc_deg_body(dst_hbm, z128, ones_hbm, deg_out,
                 dst_g, ones_v, deg_sh):
    c = lax.axis_index("c")
    s = lax.axis_index("s")
    wid = s * 2 + c
    r0 = s * ROWS_PER_TILE

    pltpu.sync_copy(z128.at[pl.ds(r0, ROWS_PER_TILE)],
                    deg_sh.at[pl.ds(r0, ROWS_PER_TILE)])
    pltpu.sync_copy(ones_hbm, ones_v)
    plsc.subcore_barrier()

    def group(g, carry):
        pltpu.sync_copy(dst_hbm.at[wid, pl.ds(g * GRP, GRP)], dst_g)

        def chunk(j, carry2):
            pltpu.sync_copy(ones_v, deg_sh.at[dst_g.at[j]], add=True)
            return carry2

        lax.fori_loop(0, GRP, chunk, 0)
        return carry

    lax.fori_loop(0, NGRP, group, 0)

    plsc.subcore_barrier()
    pltpu.sync_copy(deg_sh.at[pl.ds(r0, ROWS_PER_TILE)],
                    deg_out.at[c, pl.ds(r0, ROWS_PER_TILE)])


_sc_deg = pl.kernel(
    _sc_deg_body,
    out_type=(jax.ShapeDtypeStruct((2, NPAD, D), jnp.float32),),
    mesh=_MESH,
    scratch_types=[
        pltpu.VMEM((GRP, CHUNK), jnp.int32),
        pltpu.VMEM((CHUNK, D), jnp.float32),
        pltpu.VMEM_SHARED((NPAD, D), jnp.float32),
    ],
    name="sage_sc_deg",
)


def _dinv_body(deg_ref, o_ref):
    deg = deg_ref[0, :, 0:1] + deg_ref[1, :, 0:1]
    o_ref[...] = 1.0 / jnp.maximum(deg, 1.0)


def _dinv(deg2):
    br = 1280
    return pl.pallas_call(
        _dinv_body,
        grid=(NPAD // br,),
        in_specs=[pl.BlockSpec((2, br, D), lambda i: (0, i, 0))],
        out_specs=pl.BlockSpec((br, 1), lambda i: (i, 0)),
        out_shape=jax.ShapeDtypeStruct((NPAD, 1), jnp.float32),
    )(deg2)


def _mm_body(relu_in, h_ref, w_ref, o_ref):
    h = h_ref[...]
    if relu_in:
        h = jnp.maximum(h, 0.0)
    o_ref[...] = jnp.dot(h, w_ref[...], preferred_element_type=jnp.float32)


def _mm(h_pad, w, relu_in):
    br = 1280
    return pl.pallas_call(
        functools.partial(_mm_body, relu_in),
        grid=(NPAD // br,),
        in_specs=[
            pl.BlockSpec((br, D), lambda i: (i, 0)),
            pl.BlockSpec((D, D), lambda i: (0, 0)),
        ],
        out_specs=pl.BlockSpec((br, D), lambda i: (i, 0)),
        out_shape=jax.ShapeDtypeStruct((NPAD, D), jnp.float32),
    )(h_pad, w)


def _combine_body(relu_in, br, h_ref, w_ref, b_ref, agg_ref, dinv_ref, o_ref):
    h = h_ref[...]
    if relu_in:
        h = jnp.maximum(h, 0.0)
    s = jnp.dot(h, w_ref[...], preferred_element_type=jnp.float32) + b_ref[...]
    agg = agg_ref[0] + agg_ref[1]
    out = s + dinv_ref[...] * agg
    row = (pl.program_id(0) * br
           + lax.broadcasted_iota(jnp.int32, (br, 1), 0))
    o_ref[...] = jnp.where(row < N, out, 0.0)


def _combine(h_pad, w, b, agg2, dinv, relu_in):
    br = 1280
    return pl.pallas_call(
        functools.partial(_combine_body, relu_in, br),
        grid=(NPAD // br,),
        in_specs=[
            pl.BlockSpec((br, D), lambda i: (i, 0)),
            pl.BlockSpec((D, D), lambda i: (0, 0)),
            pl.BlockSpec((1, D), lambda i: (0, 0)),
            pl.BlockSpec((2, br, D), lambda i: (0, i, 0)),
            pl.BlockSpec((br, 1), lambda i: (i, 0)),
        ],
        out_specs=pl.BlockSpec((br, D), lambda i: (i, 0)),
        out_shape=jax.ShapeDtypeStruct((NPAD, D), jnp.float32),
    )(h_pad, w, b, agg2, dinv)


def kernel(inputs, edge_index, W_self0, W_neigh0, b0, W_self1, W_neigh1, b1,
           W_self2, W_neigh2, b2):
    src = edge_index[0].astype(jnp.int32)
    dst = edge_index[1].astype(jnp.int32)
    pad = jnp.full((EPAD - E,), N, jnp.int32)
    src3 = jnp.concatenate([src, pad]).reshape(NW, CHUNKS, CHUNK)
    dst3 = jnp.concatenate([dst, pad]).reshape(NW, CHUNKS, CHUNK)

    h0 = jnp.concatenate(
        [inputs, jnp.zeros((NPAD - N, D), jnp.float32)], axis=0)
    z128 = jnp.zeros((NPAD, D), jnp.float32)
    ones128 = jnp.ones((CHUNK, D), jnp.float32)
    b0r = b0.reshape(1, D)
    b1r = b1.reshape(1, D)
    b2r = b2.reshape(1, D)

    (deg2,) = _sc_deg(dst3, z128, ones128)
    dinv = _dinv(deg2)

    p0 = _mm(h0, W_neigh0, relu_in=False)
    (agg0,) = _sc_agg(p0, src3, dst3, z128)
    pre0 = _combine(h0, W_self0, b0r, agg0, dinv, relu_in=False)

    p1 = _mm(pre0, W_neigh1, relu_in=True)
    (agg1,) = _sc_agg(p1, src3, dst3, z128)
    pre1 = _combine(pre0, W_self1, b1r, agg1, dinv, relu_in=True)

    p2 = _mm(pre1, W_neigh2, relu_in=True)
    (agg2,) = _sc_agg(p2, src3, dst3, z128)
    pre2 = _combine(pre1, W_self2, b2r, agg2, dinv, relu_in=True)

    return (pre2[:N], pre0[:N], pre1[:N])


# speedup vs baseline: 3.3670x; 1.0812x over previous
import functools

import jax
import jax.numpy as jnp
from jax import lax
from jax.experimental import pallas as pl
from jax.experimental.pallas import tpu as pltpu, tpu_sc as plsc

N = 10000
E = 320000
D = 128
NPAD = 10240
NW = 32
CHUNK = 128
CHUNKS = 80
EPAD = NW * CHUNKS * CHUNK
ROWS_PER_TILE = NPAD // 16
DEGW = 16

_MESH = plsc.VectorSubcoreMesh(core_axis_name="c", subcore_axis_name="s")

GRP = 8
NGRP = CHUNKS // GRP
TOT_CHUNKS = EPAD // CHUNK

CH_FAST = 128
CH_SLOW = 32


def _sc_agg_body(p_hbm, src_hbm, dst_hbm, z128, agg_out,
                 src_g, dst_g, rows0, rows1, agg_sh, sem0, sem1):
    c = lax.axis_index("c")
    s = lax.axis_index("s")
    r0 = s * ROWS_PER_TILE

    base = lax.select(c == 0, s * CH_FAST, 16 * CH_FAST + s * CH_SLOW)
    ngrp = lax.select(c == 0, CH_FAST // GRP, CH_SLOW // GRP)

    pltpu.sync_copy(z128.at[pl.ds(r0, ROWS_PER_TILE)],
                    agg_sh.at[pl.ds(r0, ROWS_PER_TILE)])
    plsc.subcore_barrier()

    def group(g, carry):
        c0 = base + g * GRP
        pltpu.sync_copy(src_hbm.at[pl.ds(c0, GRP)], src_g)
        pltpu.sync_copy(dst_hbm.at[pl.ds(c0, GRP)], dst_g)
        pltpu.async_copy(p_hbm.at[src_g.at[0]], rows0, sem0)

        def pair(j, carry2):
            a = 2 * j
            b = a + 1
            pltpu.async_copy(p_hbm.at[src_g.at[b]], rows1, sem1)
            pltpu.make_async_copy(p_hbm.at[src_g.at[a]], rows0, sem0).wait()
            pltpu.sync_copy(rows0, agg_sh.at[dst_g.at[a]], add=True)

            @pl.when(j < GRP // 2 - 1)
            def _():
                pltpu.async_copy(p_hbm.at[src_g.at[a + 2]], rows0, sem0)

            pltpu.make_async_copy(p_hbm.at[src_g.at[b]], rows1, sem1).wait()
            pltpu.sync_copy(rows1, agg_sh.at[dst_g.at[b]], add=True)
            return carry2

        lax.fori_loop(0, GRP // 2, pair, 0)
        return carry

    lax.fori_loop(0, ngrp, group, 0)

    plsc.subcore_barrier()

    pltpu.sync_copy(agg_sh.at[pl.ds(r0, ROWS_PER_TILE)],
                    agg_out.at[c, pl.ds(r0, ROWS_PER_TILE)])


_sc_agg = pl.kernel(
    _sc_agg_body,
    out_type=(jax.ShapeDtypeStruct((2, NPAD, D), jnp.float32),),
    mesh=_MESH,
    scratch_types=[
        pltpu.VMEM((GRP, CHUNK), jnp.int32),
        pltpu.VMEM((GRP, CHUNK), jnp.int32),
        pltpu.VMEM((CHUNK, D), jnp.float32),
        pltpu.VMEM((CHUNK, D), jnp.float32),
        pltpu.VMEM_SHARED((NPAD, D), jnp.float32),
        pltpu.SemaphoreType.DMA,
        pltpu.SemaphoreType.DMA,
    ],
    name="sage_sc_agg",
)


def _sc_deg_body(dst_hbm, z128, ones_hbm, deg_out,
                 dst_g, ones_v, deg_sh):
    c = lax.axis_index("c")
    s = lax.axis_index("s")
    wid = s * 2 + c
    r0 = s * ROWS_PER_TILE

    pltpu.sync_copy(z128.at[pl.ds(r0, ROWS_PER_TILE)],
                    deg_sh.at[pl.ds(r0, ROWS_PER_TILE)])
    pltpu.sync_copy(ones_hbm, ones_v)
    plsc.subcore_barrier()

    def group(g, carry):
        pltpu.sync_copy(dst_hbm.at[pl.ds(wid * CHUNKS + g * GRP, GRP)], dst_g)

        def chunk(j, carry2):
            pltpu.sync_copy(ones_v, deg_sh.at[dst_g.at[j]], add=True)
            return carry2

        lax.fori_loop(0, GRP, chunk, 0)
        return carry

    lax.fori_loop(0, NGRP, group, 0)

    plsc.subcore_barrier()
    pltpu.sync_copy(deg_sh.at[pl.ds(r0, ROWS_PER_TILE)],
                    deg_out.at[c, pl.ds(r0, ROWS_PER_TILE)])


_sc_deg = pl.kernel(
    _sc_deg_body,
    out_type=(jax.ShapeDtypeStruct((2, NPAD, D), jnp.float32),),
    mesh=_MESH,
    scratch_types=[
        pltpu.VMEM((GRP, CHUNK), jnp.int32),
        pltpu.VMEM((CHUNK, D), jnp.float32),
        pltpu.VMEM_SHARED((NPAD, D), jnp.float32),
    ],
    name="sage_sc_deg",
)


def _dinv_body(deg_ref, o_ref):
    deg = deg_ref[0, :, 0:1] + deg_ref[1, :, 0:1]
    o_ref[...] = 1.0 / jnp.maximum(deg, 1.0)


def _dinv(deg2):
    br = 1280
    return pl.pallas_call(
        _dinv_body,
        grid=(NPAD // br,),
        in_specs=[pl.BlockSpec((2, br, D), lambda i: (0, i, 0))],
        out_specs=pl.BlockSpec((br, 1), lambda i: (i, 0)),
        out_shape=jax.ShapeDtypeStruct((NPAD, 1), jnp.float32),
    )(deg2)


def _mm_body(relu_in, h_ref, w_ref, o_ref):
    h = h_ref[...]
    if relu_in:
        h = jnp.maximum(h, 0.0)
    o_ref[...] = jnp.dot(h, w_ref[...], preferred_element_type=jnp.float32)


def _mm(h_pad, w, relu_in):
    br = 1280
    return pl.pallas_call(
        functools.partial(_mm_body, relu_in),
        grid=(NPAD // br,),
        in_specs=[
            pl.BlockSpec((br, D), lambda i: (i, 0)),
            pl.BlockSpec((D, D), lambda i: (0, 0)),
        ],
        out_specs=pl.BlockSpec((br, D), lambda i: (i, 0)),
        out_shape=jax.ShapeDtypeStruct((NPAD, D), jnp.float32),
    )(h_pad, w)


def _combine_body(relu_in, br, h_ref, w_ref, b_ref, agg_ref, dinv_ref, o_ref):
    h = h_ref[...]
    if relu_in:
        h = jnp.maximum(h, 0.0)
    s = jnp.dot(h, w_ref[...], preferred_element_type=jnp.float32) + b_ref[...]
    agg = agg_ref[0] + agg_ref[1]
    out = s + dinv_ref[...] * agg
    row = (pl.program_id(0) * br
           + lax.broadcasted_iota(jnp.int32, (br, 1), 0))
    o_ref[...] = jnp.where(row < N, out, 0.0)


def _combine(h_pad, w, b, agg2, dinv, relu_in):
    br = 1280
    return pl.pallas_call(
        functools.partial(_combine_body, relu_in, br),
        grid=(NPAD // br,),
        in_specs=[
            pl.BlockSpec((br, D), lambda i: (i, 0)),
            pl.BlockSpec((D, D), lambda i: (0, 0)),
            pl.BlockSpec((1, D), lambda i: (0, 0)),
            pl.BlockSpec((2, br, D), lambda i: (0, i, 0)),
            pl.BlockSpec((br, 1), lambda i: (i, 0)),
        ],
        out_specs=pl.BlockSpec((br, D), lambda i: (i, 0)),
        out_shape=jax.ShapeDtypeStruct((NPAD, D), jnp.float32),
    )(h_pad, w, b, agg2, dinv)


def kernel(inputs, edge_index, W_self0, W_neigh0, b0, W_self1, W_neigh1, b1,
           W_self2, W_neigh2, b2):
    src = edge_index[0].astype(jnp.int32)
    dst = edge_index[1].astype(jnp.int32)
    pad = jnp.full((EPAD - E,), N, jnp.int32)
    src3 = jnp.concatenate([src, pad]).reshape(TOT_CHUNKS, CHUNK)
    dst3 = jnp.concatenate([dst, pad]).reshape(TOT_CHUNKS, CHUNK)

    h0 = jnp.concatenate(
        [inputs, jnp.zeros((NPAD - N, D), jnp.float32)], axis=0)
    z128 = jnp.zeros((NPAD, D), jnp.float32)
    ones128 = jnp.ones((CHUNK, D), jnp.float32)
    b0r = b0.reshape(1, D)
    b1r = b1.reshape(1, D)
    b2r = b2.reshape(1, D)

    (deg2,) = _sc_deg(dst3, z128, ones128)
    dinv = _dinv(deg2)

    p0 = _mm(h0, W_neigh0, relu_in=False)
    (agg0,) = _sc_agg(p0, src3, dst3, z128)
    pre0 = _combine(h0, W_self0, b0r, agg0, dinv, relu_in=False)

    p1 = _mm(pre0, W_neigh1, relu_in=True)
    (agg1,) = _sc_agg(p1, src3, dst3, z128)
    pre1 = _combine(pre0, W_self1, b1r, agg1, dinv, relu_in=True)

    p2 = _mm(pre1, W_neigh2, relu_in=True)
    (agg2,) = _sc_agg(p2, src3, dst3, z128)
    pre2 = _combine(pre1, W_self2, b2r, agg2, dinv, relu_in=True)

    return (pre2[:N], pre0[:N], pre1[:N])


# speedup vs baseline: 3.7584x; 1.1163x over previous
import functools

import jax
import jax.numpy as jnp
from jax import lax
from jax.experimental import pallas as pl
from jax.experimental.pallas import tpu as pltpu, tpu_sc as plsc

N = 10000
E = 320000
D = 128
NPAD = 10240
NW = 32
CHUNK = 64
EPAD = 327680
ROWS_PER_TILE = NPAD // 16
DEGW = 16

_MESH = plsc.VectorSubcoreMesh(core_axis_name="c", subcore_axis_name="s")

GRP = 32
NBUF = 4
TOT_CHUNKS = EPAD // CHUNK
CHUNKS_PER_TILE = TOT_CHUNKS // NW
NGRP = CHUNKS_PER_TILE // GRP

CH_FAST = 256
CH_SLOW = 64


def _sc_agg_body(p_hbm, src_hbm, dst_hbm, z128, agg_out,
                 src_g, dst_g, rows0, rows1, rows2, rows3, agg_sh,
                 sem0, sem1, sem2, sem3):
    rows = [rows0, rows1, rows2, rows3]
    sems = [sem0, sem1, sem2, sem3]
    c = lax.axis_index("c")
    s = lax.axis_index("s")
    r0 = s * ROWS_PER_TILE

    base = lax.select(c == 0, s * CH_FAST, 16 * CH_FAST + s * CH_SLOW)
    ngrp = lax.select(c == 0, CH_FAST // GRP, CH_SLOW // GRP)

    pltpu.sync_copy(z128.at[pl.ds(r0, ROWS_PER_TILE)],
                    agg_sh.at[pl.ds(r0, ROWS_PER_TILE)])
    plsc.subcore_barrier()

    def group(g, carry):
        c0 = base + g * GRP
        pltpu.sync_copy(src_hbm.at[pl.ds(c0, GRP)], src_g)
        pltpu.sync_copy(dst_hbm.at[pl.ds(c0, GRP)], dst_g)
        for b in range(NBUF):
            pltpu.async_copy(p_hbm.at[src_g.at[b]], rows[b], sems[b])

        def macro(m, carry2):
            for b in range(NBUF):
                k = m * NBUF + b
                pltpu.make_async_copy(
                    p_hbm.at[src_g.at[k]], rows[b], sems[b]).wait()
                pltpu.sync_copy(rows[b], agg_sh.at[dst_g.at[k]], add=True)

                @pl.when(k + NBUF < GRP)
                def _():
                    pltpu.async_copy(
                        p_hbm.at[src_g.at[k + NBUF]], rows[b], sems[b])
            return carry2

        lax.fori_loop(0, GRP // NBUF, macro, 0)
        return carry

    lax.fori_loop(0, ngrp, group, 0)

    plsc.subcore_barrier()

    pltpu.sync_copy(agg_sh.at[pl.ds(r0, ROWS_PER_TILE)],
                    agg_out.at[c, pl.ds(r0, ROWS_PER_TILE)])


_sc_agg = pl.kernel(
    _sc_agg_body,
    out_type=(jax.ShapeDtypeStruct((2, NPAD, D), jnp.float32),),
    mesh=_MESH,
    scratch_types=[
        pltpu.VMEM((GRP, CHUNK), jnp.int32),
        pltpu.VMEM((GRP, CHUNK), jnp.int32),
        pltpu.VMEM((CHUNK, D), jnp.float32),
        pltpu.VMEM((CHUNK, D), jnp.float32),
        pltpu.VMEM((CHUNK, D), jnp.float32),
        pltpu.VMEM((CHUNK, D), jnp.float32),
        pltpu.VMEM_SHARED((NPAD, D), jnp.float32),
        pltpu.SemaphoreType.DMA,
        pltpu.SemaphoreType.DMA,
        pltpu.SemaphoreType.DMA,
        pltpu.SemaphoreType.DMA,
    ],
    name="sage_sc_agg",
)


def _sc_deg_body(dst_hbm, z128, ones_hbm, deg_out,
                 dst_g, ones_v, deg_sh):
    c = lax.axis_index("c")
    s = lax.axis_index("s")
    wid = s * 2 + c
    r0 = s * ROWS_PER_TILE

    pltpu.sync_copy(z128.at[pl.ds(r0, ROWS_PER_TILE)],
                    deg_sh.at[pl.ds(r0, ROWS_PER_TILE)])
    pltpu.sync_copy(ones_hbm, ones_v)
    plsc.subcore_barrier()

    def group(g, carry):
        pltpu.sync_copy(
            dst_hbm.at[pl.ds(wid * CHUNKS_PER_TILE + g * GRP, GRP)], dst_g)

        def chunk(j, carry2):
            pltpu.sync_copy(ones_v, deg_sh.at[dst_g.at[j]], add=True)
            return carry2

        lax.fori_loop(0, GRP, chunk, 0)
        return carry

    lax.fori_loop(0, NGRP, group, 0)

    plsc.subcore_barrier()
    pltpu.sync_copy(deg_sh.at[pl.ds(r0, ROWS_PER_TILE)],
                    deg_out.at[c, pl.ds(r0, ROWS_PER_TILE)])


_sc_deg = pl.kernel(
    _sc_deg_body,
    out_type=(jax.ShapeDtypeStruct((2, NPAD, D), jnp.float32),),
    mesh=_MESH,
    scratch_types=[
        pltpu.VMEM((GRP, CHUNK), jnp.int32),
        pltpu.VMEM((CHUNK, D), jnp.float32),
        pltpu.VMEM_SHARED((NPAD, D), jnp.float32),
    ],
    name="sage_sc_deg",
)


def _dinv_body(deg_ref, o_ref):
    deg = deg_ref[0, :, 0:1] + deg_ref[1, :, 0:1]
    o_ref[...] = 1.0 / jnp.maximum(deg, 1.0)


def _dinv(deg2):
    br = 1280
    return pl.pallas_call(
        _dinv_body,
        grid=(NPAD // br,),
        in_specs=[pl.BlockSpec((2, br, D), lambda i: (0, i, 0))],
        out_specs=pl.BlockSpec((br, 1), lambda i: (i, 0)),
        out_shape=jax.ShapeDtypeStruct((NPAD, 1), jnp.float32),
    )(deg2)


def _mm_body(relu_in, h_ref, w_ref, o_ref):
    h = h_ref[...]
    if relu_in:
        h = jnp.maximum(h, 0.0)
    o_ref[...] = jnp.dot(h, w_ref[...], preferred_element_type=jnp.float32)


def _mm(h_pad, w, relu_in):
    br = 1280
    return pl.pallas_call(
        functools.partial(_mm_body, relu_in),
        grid=(NPAD // br,),
        in_specs=[
            pl.BlockSpec((br, D), lambda i: (i, 0)),
            pl.BlockSpec((D, D), lambda i: (0, 0)),
        ],
        out_specs=pl.BlockSpec((br, D), lambda i: (i, 0)),
        out_shape=jax.ShapeDtypeStruct((NPAD, D), jnp.float32),
    )(h_pad, w)


def _combine_body(relu_in, br, h_ref, w_ref, b_ref, agg_ref, dinv_ref, o_ref):
    h = h_ref[...]
    if relu_in:
        h = jnp.maximum(h, 0.0)
    s = jnp.dot(h, w_ref[...], preferred_element_type=jnp.float32) + b_ref[...]
    agg = agg_ref[0] + agg_ref[1]
    out = s + dinv_ref[...] * agg
    row = (pl.program_id(0) * br
           + lax.broadcasted_iota(jnp.int32, (br, 1), 0))
    o_ref[...] = jnp.where(row < N, out, 0.0)


def _combine(h_pad, w, b, agg2, dinv, relu_in):
    br = 1280
    return pl.pallas_call(
        functools.partial(_combine_body, relu_in, br),
        grid=(NPAD // br,),
        in_specs=[
            pl.BlockSpec((br, D), lambda i: (i, 0)),
            pl.BlockSpec((D, D), lambda i: (0, 0)),
            pl.BlockSpec((1, D), lambda i: (0, 0)),
            pl.BlockSpec((2, br, D), lambda i: (0, i, 0)),
            pl.BlockSpec((br, 1), lambda i: (i, 0)),
        ],
        out_specs=pl.BlockSpec((br, D), lambda i: (i, 0)),
        out_shape=jax.ShapeDtypeStruct((NPAD, D), jnp.float32),
    )(h_pad, w, b, agg2, dinv)


def kernel(inputs, edge_index, W_self0, W_neigh0, b0, W_self1, W_neigh1, b1,
           W_self2, W_neigh2, b2):
    src = edge_index[0].astype(jnp.int32)
    dst = edge_index[1].astype(jnp.int32)
    pad = jnp.full((EPAD - E,), N, jnp.int32)
    src3 = jnp.concatenate([src, pad]).reshape(TOT_CHUNKS, CHUNK)
    dst3 = jnp.concatenate([dst, pad]).reshape(TOT_CHUNKS, CHUNK)

    h0 = jnp.concatenate(
        [inputs, jnp.zeros((NPAD - N, D), jnp.float32)], axis=0)
    z128 = jnp.zeros((NPAD, D), jnp.float32)
    ones128 = jnp.ones((CHUNK, D), jnp.float32)
    b0r = b0.reshape(1, D)
    b1r = b1.reshape(1, D)
    b2r = b2.reshape(1, D)

    (deg2,) = _sc_deg(dst3, z128, ones128)
    dinv = _dinv(deg2)

    p0 = _mm(h0, W_neigh0, relu_in=False)
    (agg0,) = _sc_agg(p0, src3, dst3, z128)
    pre0 = _combine(h0, W_self0, b0r, agg0, dinv, relu_in=False)

    p1 = _mm(pre0, W_neigh1, relu_in=True)
    (agg1,) = _sc_agg(p1, src3, dst3, z128)
    pre1 = _combine(pre0, W_self1, b1r, agg1, dinv, relu_in=True)

    p2 = _mm(pre1, W_neigh2, relu_in=True)
    (agg2,) = _sc_agg(p2, src3, dst3, z128)
    pre2 = _combine(pre1, W_self2, b2r, agg2, dinv, relu_in=True)

    return (pre2[:N], pre0[:N], pre1[:N])


# speedup vs baseline: 4.1182x; 1.0957x over previous
import functools

import jax
import jax.numpy as jnp
from jax import lax
from jax.experimental import pallas as pl
from jax.experimental.pallas import tpu as pltpu, tpu_sc as plsc

N = 10000
E = 320000
D = 128
NPAD = 10240
NW = 32
CHUNK = 64
EPAD = 327680
ROWS_PER_TILE = NPAD // 16
DEGW = 16

_MESH = plsc.VectorSubcoreMesh(core_axis_name="c", subcore_axis_name="s")

GRP = 32
NBUF = 4
TOT_CHUNKS = EPAD // CHUNK
CHUNKS_PER_TILE = TOT_CHUNKS // NW
NGRP = CHUNKS_PER_TILE // GRP

CH_FAST = 288
CH_SLOW = 32


def _sc_agg_body(p_hbm, src_hbm, dst_hbm, z128, agg_out,
                 src_g, dst_g, rows0, rows1, rows2, rows3, agg_sh,
                 sem0, sem1, sem2, sem3):
    rows = [rows0, rows1, rows2, rows3]
    sems = [sem0, sem1, sem2, sem3]
    c = lax.axis_index("c")
    s = lax.axis_index("s")
    r0 = s * ROWS_PER_TILE

    base = lax.select(c == 0, s * CH_FAST, 16 * CH_FAST + s * CH_SLOW)
    ngrp = lax.select(c == 0, CH_FAST // GRP, CH_SLOW // GRP)

    pltpu.sync_copy(z128.at[pl.ds(r0, ROWS_PER_TILE)],
                    agg_sh.at[pl.ds(r0, ROWS_PER_TILE)])
    plsc.subcore_barrier()

    def group(g, carry):
        c0 = base + g * GRP
        pltpu.sync_copy(src_hbm.at[pl.ds(c0, GRP)], src_g)
        pltpu.sync_copy(dst_hbm.at[pl.ds(c0, GRP)], dst_g)
        for b in range(NBUF):
            pltpu.async_copy(p_hbm.at[src_g.at[b]], rows[b], sems[b])

        def macro(m, carry2):
            for b in range(NBUF):
                k = m * NBUF + b
                pltpu.make_async_copy(
                    p_hbm.at[src_g.at[k]], rows[b], sems[b]).wait()
                pltpu.sync_copy(rows[b], agg_sh.at[dst_g.at[k]], add=True)

                @pl.when(k + NBUF < GRP)
                def _():
                    pltpu.async_copy(
                        p_hbm.at[src_g.at[k + NBUF]], rows[b], sems[b])
            return carry2

        lax.fori_loop(0, GRP // NBUF, macro, 0)
        return carry

    lax.fori_loop(0, ngrp, group, 0)

    plsc.subcore_barrier()

    pltpu.sync_copy(agg_sh.at[pl.ds(r0, ROWS_PER_TILE)],
                    agg_out.at[c, pl.ds(r0, ROWS_PER_TILE)])


_sc_agg = pl.kernel(
    _sc_agg_body,
    out_type=(jax.ShapeDtypeStruct((2, NPAD, D), jnp.float32),),
    mesh=_MESH,
    scratch_types=[
        pltpu.VMEM((GRP, CHUNK), jnp.int32),
        pltpu.VMEM((GRP, CHUNK), jnp.int32),
        pltpu.VMEM((CHUNK, D), jnp.float32),
        pltpu.VMEM((CHUNK, D), jnp.float32),
        pltpu.VMEM((CHUNK, D), jnp.float32),
        pltpu.VMEM((CHUNK, D), jnp.float32),
        pltpu.VMEM_SHARED((NPAD, D), jnp.float32),
        pltpu.SemaphoreType.DMA,
        pltpu.SemaphoreType.DMA,
        pltpu.SemaphoreType.DMA,
        pltpu.SemaphoreType.DMA,
    ],
    name="sage_sc_agg",
)


def _sc_deg_body(dst_hbm, z128, ones_hbm, deg_out,
                 dst_g, ones_v, deg_sh):
    c = lax.axis_index("c")
    s = lax.axis_index("s")
    wid = s * 2 + c
    r0 = s * ROWS_PER_TILE

    pltpu.sync_copy(z128.at[pl.ds(r0, ROWS_PER_TILE)],
                    deg_sh.at[pl.ds(r0, ROWS_PER_TILE)])
    pltpu.sync_copy(ones_hbm, ones_v)
    plsc.subcore_barrier()

    def group(g, carry):
        pltpu.sync_copy(
            dst_hbm.at[pl.ds(wid * CHUNKS_PER_TILE + g * GRP, GRP)], dst_g)

        def chunk(j, carry2):
            pltpu.sync_copy(ones_v, deg_sh.at[dst_g.at[j]], add=True)
            return carry2

        lax.fori_loop(0, GRP, chunk, 0)
        return carry

    lax.fori_loop(0, NGRP, group, 0)

    plsc.subcore_barrier()
    pltpu.sync_copy(deg_sh.at[pl.ds(r0, ROWS_PER_TILE)],
                    deg_out.at[c, pl.ds(r0, ROWS_PER_TILE)])


_sc_deg = pl.kernel(
    _sc_deg_body,
    out_type=(jax.ShapeDtypeStruct((2, NPAD, D), jnp.float32),),
    mesh=_MESH,
    scratch_types=[
        pltpu.VMEM((GRP, CHUNK), jnp.int32),
        pltpu.VMEM((CHUNK, D), jnp.float32),
        pltpu.VMEM_SHARED((NPAD, D), jnp.float32),
    ],
    name="sage_sc_deg",
)


def _dinv_body(deg_ref, o_ref):
    deg = deg_ref[0, :, 0:1] + deg_ref[1, :, 0:1]
    o_ref[...] = 1.0 / jnp.maximum(deg, 1.0)


def _dinv(deg2):
    br = 1280
    return pl.pallas_call(
        _dinv_body,
        grid=(NPAD // br,),
        in_specs=[pl.BlockSpec((2, br, D), lambda i: (0, i, 0))],
        out_specs=pl.BlockSpec((br, 1), lambda i: (i, 0)),
        out_shape=jax.ShapeDtypeStruct((NPAD, 1), jnp.float32),
    )(deg2)


def _mm_body(relu_in, h_ref, w_ref, o_ref):
    h = h_ref[...]
    if relu_in:
        h = jnp.maximum(h, 0.0)
    o_ref[...] = jnp.dot(h, w_ref[...], preferred_element_type=jnp.float32)


def _mm(h_pad, w, relu_in):
    br = 1280
    return pl.pallas_call(
        functools.partial(_mm_body, relu_in),
        grid=(NPAD // br,),
        in_specs=[
            pl.BlockSpec((br, D), lambda i: (i, 0)),
            pl.BlockSpec((D, D), lambda i: (0, 0)),
        ],
        out_specs=pl.BlockSpec((br, D), lambda i: (i, 0)),
        out_shape=jax.ShapeDtypeStruct((NPAD, D), jnp.float32),
    )(h_pad, w)


def _combine_body(relu_in, br, h_ref, w_ref, b_ref, agg_ref, dinv_ref, o_ref):
    h = h_ref[...]
    if relu_in:
        h = jnp.maximum(h, 0.0)
    s = jnp.dot(h, w_ref[...], preferred_element_type=jnp.float32) + b_ref[...]
    agg = agg_ref[0] + agg_ref[1]
    out = s + dinv_ref[...] * agg
    row = (pl.program_id(0) * br
           + lax.broadcasted_iota(jnp.int32, (br, 1), 0))
    o_ref[...] = jnp.where(row < N, out, 0.0)


def _combine(h_pad, w, b, agg2, dinv, relu_in):
    br = 1280
    return pl.pallas_call(
        functools.partial(_combine_body, relu_in, br),
        grid=(NPAD // br,),
        in_specs=[
            pl.BlockSpec((br, D), lambda i: (i, 0)),
            pl.BlockSpec((D, D), lambda i: (0, 0)),
            pl.BlockSpec((1, D), lambda i: (0, 0)),
            pl.BlockSpec((2, br, D), lambda i: (0, i, 0)),
            pl.BlockSpec((br, 1), lambda i: (i, 0)),
        ],
        out_specs=pl.BlockSpec((br, D), lambda i: (i, 0)),
        out_shape=jax.ShapeDtypeStruct((NPAD, D), jnp.float32),
    )(h_pad, w, b, agg2, dinv)


def kernel(inputs, edge_index, W_self0, W_neigh0, b0, W_self1, W_neigh1, b1,
           W_self2, W_neigh2, b2):
    src = edge_index[0].astype(jnp.int32)
    dst = edge_index[1].astype(jnp.int32)
    pad = jnp.full((EPAD - E,), N, jnp.int32)
    src3 = jnp.concatenate([src, pad]).reshape(TOT_CHUNKS, CHUNK)
    dst3 = jnp.concatenate([dst, pad]).reshape(TOT_CHUNKS, CHUNK)

    h0 = jnp.concatenate(
        [inputs, jnp.zeros((NPAD - N, D), jnp.float32)], axis=0)
    z128 = jnp.zeros((NPAD, D), jnp.float32)
    ones128 = jnp.ones((CHUNK, D), jnp.float32)
    b0r = b0.reshape(1, D)
    b1r = b1.reshape(1, D)
    b2r = b2.reshape(1, D)

    (deg2,) = _sc_deg(dst3, z128, ones128)
    dinv = _dinv(deg2)

    p0 = _mm(h0, W_neigh0, relu_in=False)
    (agg0,) = _sc_agg(p0, src3, dst3, z128)
    pre0 = _combine(h0, W_self0, b0r, agg0, dinv, relu_in=False)

    p1 = _mm(pre0, W_neigh1, relu_in=True)
    (agg1,) = _sc_agg(p1, src3, dst3, z128)
    pre1 = _combine(pre0, W_self1, b1r, agg1, dinv, relu_in=True)

    p2 = _mm(pre1, W_neigh2, relu_in=True)
    (agg2,) = _sc_agg(p2, src3, dst3, z128)
    pre2 = _combine(pre1, W_self2, b2r, agg2, dinv, relu_in=True)

    return (pre2[:N], pre0[:N], pre1[:N])
